# baseline (speedup 1.0000x reference)
import jax
import jax.numpy as jnp
from jax.experimental import pallas as pl

_TILE_B = 512
_K = 1024


def _fused_kernel(xt_ref, we1_ref, be1_ref, we2_ref, be2_ref, we3_ref,
                  be3_ref, cb_ref, cbt_ref, wq1_ref, bq1_ref, wq2_ref,
                  bq2_ref, wq3_ref, bq3_ref, out_ref):
    h = jax.lax.dot_general(xt_ref[...], we1_ref[...],
                            (((0,), (0,)), ((), ())),
                            preferred_element_type=jnp.float32)
    h = jnp.maximum(h + be1_ref[...], 0.0)
    h = jnp.maximum(jnp.dot(h, we2_ref[...],
                            preferred_element_type=jnp.float32) + be2_ref[...], 0.0)
    z = jnp.dot(h, we3_ref[...], preferred_element_type=jnp.float32) + be3_ref[...]

    cbt = cbt_ref[...]
    cnorm = jnp.sum(cbt * cbt, axis=0, keepdims=True)
    zc = jnp.dot(z, cbt, preferred_element_type=jnp.float32)
    d = cnorm - 2.0 * zc

    iota = jax.lax.broadcasted_iota(jnp.int32, d.shape, 1)
    dmin = jnp.min(d, axis=1, keepdims=True)
    idx = jnp.min(jnp.where(d == dmin, iota, _K), axis=1, keepdims=True)
    onehot = (iota == idx).astype(jnp.float32)
    z_q = jnp.dot(onehot, cb_ref[...], preferred_element_type=jnp.float32)

    e = jnp.tanh(jnp.dot(z_q, wq1_ref[...],
                         preferred_element_type=jnp.float32) + bq1_ref[...])
    e = jnp.tanh(jnp.dot(e, wq2_ref[...],
                         preferred_element_type=jnp.float32) + bq2_ref[...])
    out_ref[...] = jnp.dot(e, wq3_ref[...],
                           preferred_element_type=jnp.float32) + bq3_ref[...]


@jax.jit
def kernel(x, We1, be1, We2, be2, We3, be3, codebook,
           Wq1, bq1, Wq2, bq2, Wq3, bq3):
    B, in_dim = x.shape
    out_dims = Wq3.shape[1]
    nb = B // _TILE_B
    xt = x.T

    def full(a):
        return pl.BlockSpec(a.shape, lambda i: (0,) * a.ndim)

    return pl.pallas_call(
        _fused_kernel,
        grid=(nb,),
        in_specs=[
            pl.BlockSpec((in_dim, _TILE_B), lambda i: (0, i)),
            full(We1), full(be1), full(We2), full(be2), full(We3), full(be3),
            full(codebook), full(codebook.T),
            full(Wq1), full(bq1), full(Wq2), full(bq2), full(Wq3), full(bq3),
        ],
        out_specs=pl.BlockSpec((_TILE_B, out_dims), lambda i: (i, 0)),
        out_shape=jax.ShapeDtypeStruct((B, out_dims), jnp.float32),
    )(xt, We1, be1, We2, be2, We3, be3, codebook, codebook.T,
      Wq1, bq1, Wq2, bq2, Wq3, bq3)
